# baseline (speedup 1.0000x reference)
import functools

import jax
import jax.numpy as jnp
from jax import lax
from jax.experimental import pallas as pl
from jax.experimental.pallas import tpu as pltpu
from jax.experimental.pallas import tpu_sc as plsc

BATCH = 16384
FEAT = 64
HID1 = 128
HID2 = 64

NC, NS = 2, 16
NW = NC * NS
BPW = BATCH // NW
CHUNK = 128
NCHUNK = BPW // CHUNK


def _gather_body(user_table, video_table, uidx, vidx, out_u, out_v,
                 idx_u, idx_v, rows_u, rows_v, sem):
    wid = lax.axis_index("s") * NC + lax.axis_index("c")
    row0 = wid * NCHUNK
    base = wid * BPW
    pltpu.sync_copy(uidx.at[pl.ds(row0, NCHUNK)], idx_u)
    pltpu.sync_copy(vidx.at[pl.ds(row0, NCHUNK)], idx_v)
    copies = []
    for j in range(NCHUNK):
        copies.append(pltpu.async_copy(
            user_table.at[idx_u.at[j]], rows_u.at[pl.ds(j * CHUNK, CHUNK)], sem))
        copies.append(pltpu.async_copy(
            video_table.at[idx_v.at[j]], rows_v.at[pl.ds(j * CHUNK, CHUNK)], sem))
    for c in copies:
        c.wait()
    pltpu.sync_copy(rows_u, out_u.at[pl.ds(base, BPW)])
    pltpu.sync_copy(rows_v, out_v.at[pl.ds(base, BPW)])


_gather = pl.kernel(
    _gather_body,
    out_type=(jax.ShapeDtypeStruct((BATCH, FEAT), jnp.float32),
              jax.ShapeDtypeStruct((BATCH, FEAT), jnp.float32)),
    mesh=plsc.VectorSubcoreMesh(core_axis_name="c", subcore_axis_name="s"),
    scratch_types=[
        pltpu.VMEM((NCHUNK, CHUNK), jnp.int32),
        pltpu.VMEM((NCHUNK, CHUNK), jnp.int32),
        pltpu.VMEM((BPW, FEAT), jnp.float32),
        pltpu.VMEM((BPW, FEAT), jnp.float32),
        pltpu.SemaphoreType.DMA,
    ],
    compiler_params=pltpu.CompilerParams(use_tc_tiling_on_sc=False),
)


BLK = 1024


def _mlp_body(u_ref, v_ref, w1u_ref, w1v_ref, b1_ref, w2_ref, b2_ref,
              w3_ref, b3_ref, o_ref):
    h = jnp.dot(u_ref[...], w1u_ref[...], preferred_element_type=jnp.float32)
    h = h + jnp.dot(v_ref[...], w1v_ref[...], preferred_element_type=jnp.float32)
    h = jnp.maximum(h + b1_ref[...], 0.0)
    h = jnp.maximum(
        jnp.dot(h, w2_ref[...], preferred_element_type=jnp.float32) + b2_ref[...],
        0.0)
    z = jnp.dot(h, w3_ref[...], preferred_element_type=jnp.float32) + b3_ref[...]
    o_ref[...] = jax.nn.sigmoid(z)


_mlp = pl.pallas_call(
    _mlp_body,
    grid=(BATCH // BLK,),
    in_specs=[
        pl.BlockSpec((BLK, FEAT), lambda i: (i, 0)),
        pl.BlockSpec((BLK, FEAT), lambda i: (i, 0)),
        pl.BlockSpec((FEAT, HID1), lambda i: (0, 0)),
        pl.BlockSpec((FEAT, HID1), lambda i: (0, 0)),
        pl.BlockSpec((1, HID1), lambda i: (0, 0)),
        pl.BlockSpec((HID1, HID2), lambda i: (0, 0)),
        pl.BlockSpec((1, HID2), lambda i: (0, 0)),
        pl.BlockSpec((HID2, 1), lambda i: (0, 0)),
        pl.BlockSpec((1, 1), lambda i: (0, 0)),
    ],
    out_specs=pl.BlockSpec((BLK, 1), lambda i: (i, 0)),
    out_shape=jax.ShapeDtypeStruct((BATCH, 1), jnp.float32),
)


def kernel(user_table, video_table, W1, b1, W2, b2, W3, b3,
           user_indices, video_indices):
    uidx = user_indices.astype(jnp.int32).reshape(BATCH // CHUNK, CHUNK)
    vidx = video_indices.astype(jnp.int32).reshape(BATCH // CHUNK, CHUNK)
    u_vec, v_vec = _gather(user_table, video_table, uidx, vidx)
    return _mlp(u_vec, v_vec, W1[:FEAT], W1[FEAT:], b1.reshape(1, HID1),
                W2, b2.reshape(1, HID2), W3, b3.reshape(1, 1))
